# baseline (speedup 1.0000x reference)
import functools

import jax
import jax.numpy as jnp
from jax import lax
from jax.experimental import pallas as pl
from jax.experimental.pallas import tpu as pltpu
from jax.experimental.pallas import tpu_sc as plsc

NCL = 32
B = 16384

_info = plsc.get_sparse_core_info()
_NC, _NS, _L = _info.num_cores, _info.num_subcores, _info.num_lanes
_NW = _NC * _NS
_BPW = B // _NW
_CH = 128
_NG = _BPW // _CH


def _tec_body(ixs_hbm, w_hbm, out_hbm, idx_v, rows_v, rows_t, sem_in, sem_out):
    wid = lax.axis_index("s") * _NC + lax.axis_index("c")
    base = wid * _BPW
    pltpu.sync_copy(ixs_hbm.at[pl.ds(wid * _NG, _NG)], idx_v)

    for g in range(_NG):
        pltpu.async_copy(
            w_hbm.at[idx_v.at[g]],
            rows_v.at[pl.ds(g * _CH, _CH)],
            sem_in,
        )
    pltpu.make_async_copy(w_hbm.at[pl.ds(0, _BPW)], rows_v, sem_in).wait()

    iota_t = lax.iota(jnp.int32, _L) * _BPW

    def tr(j, carry):
        lo = rows_v[j, pl.ds(0, _L)]
        hi = rows_v[j, pl.ds(_L, _L)]
        plsc.store_scatter(rows_t, [iota_t + j], lo)
        plsc.store_scatter(rows_t, [iota_t + (j + _L * _BPW)], hi)
        return carry

    lax.fori_loop(0, _BPW, tr, 0)

    def wr(c, carry):
        pltpu.async_copy(
            rows_t.at[pl.ds(c * _BPW, _BPW)],
            out_hbm.at[c, pl.ds(base, _BPW)],
            sem_out,
        )
        return carry

    lax.fori_loop(0, NCL, wr, 0)
    pltpu.make_async_copy(out_hbm.at[0], rows_t, sem_out).wait()


def kernel(variantxgene_ixs, weight):
    f = functools.partial(
        pl.kernel,
        mesh=plsc.VectorSubcoreMesh(core_axis_name="c", subcore_axis_name="s"),
        compiler_params=pltpu.CompilerParams(
            needs_layout_passes=False, use_tc_tiling_on_sc=False
        ),
        out_type=jax.ShapeDtypeStruct((NCL, B), jnp.float32),
        scratch_types=[
            pltpu.VMEM((_NG, _CH), jnp.int32),
            pltpu.VMEM((_BPW, NCL), jnp.float32),
            pltpu.VMEM((NCL * _BPW,), jnp.float32),
            pltpu.SemaphoreType.DMA,
            pltpu.SemaphoreType.DMA,
        ],
    )(_tec_body)
    return f(variantxgene_ixs.reshape(_NW * _NG, _CH), weight)
